# baseline (speedup 1.0000x reference)
import jax
import jax.numpy as jnp
from jax import lax
from jax.experimental import pallas as pl
from jax.experimental.pallas import tpu as pltpu
from jax.experimental.pallas import tpu_sc as plsc

_B = 16384
_D = 64
_LANES = 16
_CHUNK = 128

_NC = 2
_NS = 16
_NW = _NC * _NS
_BPW = _B // _NW
_NCHUNK = _BPW // _CHUNK
_NHALF = 2
_HBPW = _BPW // _NHALF
_HCHUNK = _HBPW // _CHUNK
_NGROUP = _HBPW // _LANES


def _bmf_body(uid_hbm, iid_hbm, ut_hbm, it_hbm, ub_hbm, ib_hbm, gb_hbm,
              out_hbm,
              uidx_v, iidx_v, uh_v, ih_v, uq_v, iq_v,
              urows_v, irows_v, ubias_v, ibias_v, out_v, gb_v, sem):
    wid = lax.axis_index("s") * _NC + lax.axis_index("c")
    cbase = wid * _NCHUNK
    base = wid * _BPW

    pltpu.sync_copy(uid_hbm.at[pl.ds(cbase, _NCHUNK)], uidx_v)
    pltpu.sync_copy(iid_hbm.at[pl.ds(cbase, _NCHUNK)], iidx_v)
    pltpu.sync_copy(gb_hbm, gb_v)

    for j in range(_NCHUNK):
        for k in range(_CHUNK // _LANES):
            sl = pl.ds(k * _LANES, _LANES)
            uh_v[j, sl] = uidx_v[j, sl] >> 1
            ih_v[j, sl] = iidx_v[j, sl] >> 1
            uq_v[j, sl] = uidx_v[j, sl] >> 4
            iq_v[j, sl] = iidx_v[j, sl] >> 4

    bias_copies = []
    for j in range(_NCHUNK):
        s = j * _CHUNK
        bias_copies.append(pltpu.async_copy(
            ub_hbm.at[uq_v.at[j]], ubias_v.at[pl.ds(s, _CHUNK)], sem))
        bias_copies.append(pltpu.async_copy(
            ib_hbm.at[iq_v.at[j]], ibias_v.at[pl.ds(s, _CHUNK)], sem))

    gb = gb_v[...]

    for h in range(_NHALF):
        emb_copies = []
        for j in range(_HCHUNK):
            cj = h * _HCHUNK + j
            s = j * _CHUNK
            emb_copies.append(pltpu.async_copy(
                ut_hbm.at[uh_v.at[cj]], urows_v.at[pl.ds(s, _CHUNK)], sem))
            emb_copies.append(pltpu.async_copy(
                it_hbm.at[ih_v.at[cj]], irows_v.at[pl.ds(s, _CHUNK)], sem))
        for c in emb_copies:
            c.wait()
        if h == 0:
            for c in bias_copies:
                c.wait()

        def group(g, carry):
            p = h * _HBPW + g * _LANES + lax.iota(jnp.int32, _LANES)
            pl_local = g * _LANES + lax.iota(jnp.int32, _LANES)
            jv = p >> 7
            kv = p & 127
            uidx = plsc.load_gather(uidx_v, [jv, kv])
            iidx = plsc.load_gather(iidx_v, [jv, kv])
            ucol0 = (uidx & 1) * _D
            icol0 = (iidx & 1) * _D
            acc = jnp.zeros((_LANES,), jnp.float32)
            for d in range(_D):
                u = plsc.load_gather(urows_v, [pl_local, ucol0 + d])
                v = plsc.load_gather(irows_v, [pl_local, icol0 + d])
                acc = acc + u * v
            ub = plsc.load_gather(ubias_v, [p, uidx & 15])
            ib = plsc.load_gather(ibias_v, [p, iidx & 15])
            z = acc + ub + ib + gb
            out_v[pl.ds(h * _HBPW + g * _LANES, _LANES)] = (
                1.0 / (1.0 + jnp.exp(-z)))
            return carry

        lax.fori_loop(0, _NGROUP, group, 0)

    pltpu.sync_copy(out_v, out_hbm.at[pl.ds(base, _BPW)])


@jax.jit
def _bmf(uid, iid, ut, it, ub, ib, gb):
    mesh = plsc.VectorSubcoreMesh(core_axis_name="c", subcore_axis_name="s")
    kfn = pl.kernel(
        _bmf_body,
        mesh=mesh,
        compiler_params=pltpu.CompilerParams(
            needs_layout_passes=False, use_tc_tiling_on_sc=False),
        out_type=jax.ShapeDtypeStruct((_B,), jnp.float32),
        scratch_types=[
            pltpu.VMEM((_NCHUNK, _CHUNK), jnp.int32),
            pltpu.VMEM((_NCHUNK, _CHUNK), jnp.int32),
            pltpu.VMEM((_NCHUNK, _CHUNK), jnp.int32),
            pltpu.VMEM((_NCHUNK, _CHUNK), jnp.int32),
            pltpu.VMEM((_NCHUNK, _CHUNK), jnp.int32),
            pltpu.VMEM((_NCHUNK, _CHUNK), jnp.int32),
            pltpu.VMEM((_HBPW, 2 * _D), jnp.float32),
            pltpu.VMEM((_HBPW, 2 * _D), jnp.float32),
            pltpu.VMEM((_BPW, _LANES), jnp.float32),
            pltpu.VMEM((_BPW, _LANES), jnp.float32),
            pltpu.VMEM((_BPW,), jnp.float32),
            pltpu.VMEM((_LANES,), jnp.float32),
            pltpu.SemaphoreType.DMA,
        ],
    )
    return kfn(uid, iid, ut, it, ub, ib, gb)


def kernel(user_ids, item_ids, user_table, item_table, user_bias_table,
           item_bias_table, global_bias):
    uid = user_ids.astype(jnp.int32).reshape(_B // _CHUNK, _CHUNK)
    iid = item_ids.astype(jnp.int32).reshape(_B // _CHUNK, _CHUNK)
    ut2 = user_table.reshape(-1, 2 * _D)
    it2 = item_table.reshape(-1, 2 * _D)
    ubq = user_bias_table.reshape(-1, _LANES)
    ibq = item_bias_table.reshape(-1, _LANES)
    gb = jnp.broadcast_to(global_bias.reshape(()), (_LANES,))
    out = _bmf(uid, iid, ut2, it2, ubq, ibq, gb)
    return out.reshape(_B, 1)
